# baseline (speedup 1.0000x reference)
import functools

import jax
import jax.numpy as jnp
from jax import lax
from jax.experimental import pallas as pl
from jax.experimental.pallas import tpu as pltpu
from jax.experimental.pallas import tpu_sc as plsc


def _sc_gather(cs_hbm, cq_hbm, idx_hbm, out_cs, out_cq,
               idx_v, rows_cs, rows_cq, sem_cs, sem_cq,
               *, nc, b_per_w):
    wid = lax.axis_index("s") * nc + lax.axis_index("c")
    base = wid * b_per_w
    pltpu.sync_copy(idx_hbm.at[pl.ds(base, b_per_w)], idx_v)
    c1 = pltpu.async_copy(cs_hbm.at[idx_v], rows_cs, sem_cs)
    c2 = pltpu.async_copy(cq_hbm.at[idx_v], rows_cq, sem_cq)
    c1.wait()
    c2.wait()
    pltpu.sync_copy(rows_cs, out_cs.at[pl.ds(base, b_per_w)])
    pltpu.sync_copy(rows_cq, out_cq.at[pl.ds(base, b_per_w)])


def _tc_hash(x_ref, wt_ref, b_ref, o_ref):
    acc = jnp.dot(x_ref[...], wt_ref[...], preferred_element_type=jnp.float32)
    o_ref[...] = jnp.tanh(acc + b_ref[...])


def kernel(corp_self, corp_query, W1, b1, corp_batch):
    num_corp, hidden = corp_self.shape
    hash_dim = W1.shape[0]
    batch = corp_batch.shape[0]

    info = plsc.get_sparse_core_info()
    nc, ns = info.num_cores, info.num_subcores
    nw = nc * ns
    b_per_w = batch // nw

    mesh = plsc.VectorSubcoreMesh(core_axis_name="c", subcore_axis_name="s")
    gather = pl.kernel(
        functools.partial(_sc_gather, nc=nc, b_per_w=b_per_w),
        out_type=(
            jax.ShapeDtypeStruct((batch, hidden), jnp.float32),
            jax.ShapeDtypeStruct((batch, hidden), jnp.float32),
        ),
        mesh=mesh,
        compiler_params=pltpu.CompilerParams(use_tc_tiling_on_sc=False),
        scratch_types=[
            pltpu.VMEM((b_per_w,), jnp.int32),
            pltpu.VMEM((b_per_w, hidden), jnp.float32),
            pltpu.VMEM((b_per_w, hidden), jnp.float32),
            pltpu.SemaphoreType.DMA,
            pltpu.SemaphoreType.DMA,
        ],
    )
    cs_embs, cq_embs = gather(corp_self, corp_query, corp_batch)

    bm = 2048
    cs_hash = pl.pallas_call(
        _tc_hash,
        grid=(batch // bm,),
        in_specs=[
            pl.BlockSpec((bm, hidden), lambda i: (i, 0)),
            pl.BlockSpec((hidden, hash_dim), lambda i: (0, 0)),
            pl.BlockSpec((1, hash_dim), lambda i: (0, 0)),
        ],
        out_specs=pl.BlockSpec((bm, hash_dim), lambda i: (i, 0)),
        out_shape=jax.ShapeDtypeStruct((batch, hash_dim), jnp.float32),
    )(cs_embs, W1.T, b1.reshape(1, hash_dim))

    return (cs_hash, cq_embs)


# speedup vs baseline: 2.2781x; 2.2781x over previous
import functools

import jax
import jax.numpy as jnp
from jax import lax
from jax.experimental import pallas as pl
from jax.experimental.pallas import tpu as pltpu
from jax.experimental.pallas import tpu_sc as plsc


def _sc_gather(cs3, cq3, idx_hbm, out_cs, out_cq,
               idx_v, buf_cs, buf_cq, grp_v, sem_cs, sem_cq,
               *, nc, b_per_w, hidden):
    wid = lax.axis_index("s") * nc + lax.axis_index("c")
    base = wid * b_per_w
    half = b_per_w // 2
    pltpu.sync_copy(idx_hbm.at[pl.ds(base, b_per_w)], idx_v)

    ngroups = half // 16

    for h in range(2):
        hbase = h * half

        def group_body(gi, carry, hbase=hbase):
            iv = idx_v[pl.ds(hbase + gi * 16, 16)]
            gv = lax.shift_right_logical(iv, 3)
            sv = lax.bitwise_and(iv, 7)
            descs = []
            for lane in range(16):
                g = gv[lane]
                s = sv[lane]
                i = gi * 16 + lane
                descs.append(pltpu.async_copy(
                    cs3.at[g, s], buf_cs.at[i, pl.ds(0, hidden)], sem_cs))
                descs.append(pltpu.async_copy(
                    cq3.at[g, s], buf_cq.at[i, pl.ds(0, hidden)], sem_cq))
            for d in descs:
                d.wait()
            return carry

        lax.fori_loop(0, ngroups, group_body, 0)
        pltpu.sync_copy(buf_cs, out_cs.at[pl.ds(base + hbase, half)])
        pltpu.sync_copy(buf_cq, out_cq.at[pl.ds(base + hbase, half)])


def _tc_hash(x_ref, wt_ref, b_ref, o_ref, *, hidden):
    acc = jnp.dot(x_ref[:, :hidden], wt_ref[...],
                  preferred_element_type=jnp.float32)
    o_ref[...] = jnp.tanh(acc + b_ref[...])


def kernel(corp_self, corp_query, W1, b1, corp_batch):
    num_corp, hidden = corp_self.shape
    hash_dim = W1.shape[0]
    batch = corp_batch.shape[0]
    padded = 2 * hidden

    info = plsc.get_sparse_core_info()
    nc, ns = info.num_cores, info.num_subcores
    nw = nc * ns
    b_per_w = batch // nw
    half = b_per_w // 2

    cs3 = corp_self.reshape(num_corp // 8, 8, hidden)
    cq3 = corp_query.reshape(num_corp // 8, 8, hidden)

    mesh = plsc.VectorSubcoreMesh(core_axis_name="c", subcore_axis_name="s")
    gather = pl.kernel(
        functools.partial(_sc_gather, nc=nc, b_per_w=b_per_w, hidden=hidden),
        out_type=(
            jax.ShapeDtypeStruct((batch, padded), jnp.float32),
            jax.ShapeDtypeStruct((batch, padded), jnp.float32),
        ),
        mesh=mesh,
        scratch_types=[
            pltpu.VMEM((b_per_w,), jnp.int32),
            pltpu.VMEM((half, padded), jnp.float32),
            pltpu.VMEM((half, padded), jnp.float32),
            pltpu.VMEM((2, 8, hidden), jnp.float32),
            pltpu.SemaphoreType.DMA,
            pltpu.SemaphoreType.DMA,
        ],
    )
    cs_pad, cq_pad = gather(cs3, cq3, corp_batch)

    bm = 2048
    cs_hash = pl.pallas_call(
        functools.partial(_tc_hash, hidden=hidden),
        grid=(batch // bm,),
        in_specs=[
            pl.BlockSpec((bm, padded), lambda i: (i, 0)),
            pl.BlockSpec((hidden, hash_dim), lambda i: (0, 0)),
            pl.BlockSpec((1, hash_dim), lambda i: (0, 0)),
        ],
        out_specs=pl.BlockSpec((bm, hash_dim), lambda i: (i, 0)),
        out_shape=jax.ShapeDtypeStruct((batch, hash_dim), jnp.float32),
    )(cs_pad, W1.T, b1.reshape(1, hash_dim))

    return (cs_hash, cq_pad[:, :hidden])
